# baseline (speedup 1.0000x reference)
import jax
import jax.numpy as jnp
from jax.experimental import pallas as pl
from jax.experimental.pallas import tpu as pltpu

_N = 16 * 32 * 32
_K = 1024
_C = 64
_CH = 2048
_NCH = _N // _CH
_NBUF = 3


def _body(code_hbm, q_hbm, cb_ref, out_ref,
          code_buf, q_buf, acc_ref, code_sem, q_sem):

    def _copies(c):
        slot = c % _NBUF
        rows = pl.ds(c * _CH, _CH)
        return (
            pltpu.make_async_copy(
                code_hbm.at[rows, :], code_buf.at[slot], code_sem.at[slot]),
            pltpu.make_async_copy(
                q_hbm.at[rows, :], q_buf.at[slot], q_sem.at[slot]),
        )

    for c in range(_NBUF):
        for cp in _copies(c):
            cp.start()

    for c in range(_NCH):
        slot = c % _NBUF
        for cp in _copies(c):
            cp.wait()
        q_aug = jnp.concatenate(
            [q_buf[slot], jnp.ones((_CH, 1), dtype=jnp.float32)], axis=1)
        partial = jax.lax.dot_general(
            q_aug,
            code_buf[slot],
            dimension_numbers=(((0,), (0,)), ((), ())),
            preferred_element_type=jnp.float32,
        )
        if c == 0:
            acc_ref[...] = partial
        else:
            acc_ref[...] += partial
        if c + _NBUF < _NCH:
            for cp in _copies(c + _NBUF):
                cp.start()

    acc = acc_ref[...]
    count = acc[_C:_C + 1, :]
    mean_q = acc[:_C, :] / count
    mask = count != 0.0
    cb_t = jnp.transpose(cb_ref[...])
    sq = (cb_t - mean_q) ** 2
    sq = jnp.where(mask, sq, 0.0)
    n_selected = jnp.sum(mask.astype(jnp.float32)) * _C
    out_ref[...] = jnp.reshape(jnp.sum(sq) / n_selected, (1, 1))


def kernel(quantized, code, codebook):
    code2d = code.reshape(_N, _K)
    q2d = quantized.reshape(_N, _C)

    out = pl.pallas_call(
        _body,
        in_specs=[
            pl.BlockSpec(memory_space=pltpu.HBM),
            pl.BlockSpec(memory_space=pltpu.HBM),
            pl.BlockSpec(memory_space=pltpu.VMEM),
        ],
        out_specs=pl.BlockSpec(memory_space=pltpu.VMEM),
        out_shape=jax.ShapeDtypeStruct((1, 1), jnp.float32),
        scratch_shapes=[
            pltpu.VMEM((_NBUF, _CH, _K), jnp.float32),
            pltpu.VMEM((_NBUF, _CH, _C), jnp.float32),
            pltpu.VMEM((_C + 1, _K), jnp.float32),
            pltpu.SemaphoreType.DMA((_NBUF,)),
            pltpu.SemaphoreType.DMA((_NBUF,)),
        ],
    )(code2d, q2d, codebook)
    return out[0, 0]


# speedup vs baseline: 1.0632x; 1.0632x over previous
import jax
import jax.numpy as jnp
from jax.experimental import pallas as pl
from jax.experimental.pallas import tpu as pltpu

_N = 16 * 32 * 32
_K = 1024
_C = 64
_BLK = 2048
_NBLK = _N // _BLK


def _body(code_ref, q_ref, cb_ref, out_ref, acc_ref):
    i = pl.program_id(0)

    @pl.when(i == 0)
    def _init():
        acc_ref[...] = jnp.zeros_like(acc_ref)

    q_aug = jnp.concatenate(
        [q_ref[...], jnp.ones((_BLK, 1), dtype=jnp.float32)], axis=1)
    acc_ref[...] += jax.lax.dot_general(
        q_aug,
        code_ref[...],
        dimension_numbers=(((0,), (0,)), ((), ())),
        preferred_element_type=jnp.float32,
    )

    @pl.when(i == _NBLK - 1)
    def _epilogue():
        acc = acc_ref[...]
        count = acc[_C:_C + 1, :]
        mean_q = acc[:_C, :] / count
        mask = count != 0.0
        cb_t = jnp.transpose(cb_ref[...])
        sq = (cb_t - mean_q) ** 2
        sq = jnp.where(mask, sq, 0.0)
        n_selected = jnp.sum(mask.astype(jnp.float32)) * _C
        out_ref[...] = jnp.reshape(jnp.sum(sq) / n_selected, (1, 1))


def kernel(quantized, code, codebook):
    code2d = code.reshape(_N, _K)
    q2d = quantized.reshape(_N, _C)

    out = pl.pallas_call(
        _body,
        grid=(_NBLK,),
        in_specs=[
            pl.BlockSpec((_BLK, _K), lambda i: (i, 0)),
            pl.BlockSpec((_BLK, _C), lambda i: (i, 0)),
            pl.BlockSpec((_K, _C), lambda i: (0, 0)),
        ],
        out_specs=pl.BlockSpec((1, 1), lambda i: (0, 0)),
        out_shape=jax.ShapeDtypeStruct((1, 1), jnp.float32),
        scratch_shapes=[pltpu.VMEM((_C + 1, _K), jnp.float32)],
        compiler_params=pltpu.CompilerParams(
            dimension_semantics=("arbitrary",),
        ),
    )(code2d, q2d, codebook)
    return out[0, 0]


# speedup vs baseline: 1.0716x; 1.0079x over previous
import jax
import jax.numpy as jnp
from jax.experimental import pallas as pl
from jax.experimental.pallas import tpu as pltpu

_N = 16 * 32 * 32
_K = 1024
_C = 64
_BLK = 2048
_NBLK = _N // _BLK


def _body(code_ref, q_ref, cb_ref, out_ref, acc_ref):
    i = pl.program_id(0)

    @pl.when(i == 0)
    def _init():
        acc_ref[...] = jnp.zeros_like(acc_ref)

    q_aug = jnp.concatenate(
        [q_ref[pl.ds(i * _BLK, _BLK), :],
         jnp.ones((_BLK, 1), dtype=jnp.float32)], axis=1)
    acc_ref[...] += jax.lax.dot_general(
        q_aug,
        code_ref[...],
        dimension_numbers=(((0,), (0,)), ((), ())),
        preferred_element_type=jnp.float32,
    )

    @pl.when(i == _NBLK - 1)
    def _epilogue():
        acc = acc_ref[...]
        count = acc[_C:_C + 1, :]
        mean_q = acc[:_C, :] / count
        mask = count != 0.0
        cb_t = jnp.transpose(cb_ref[...])
        sq = (cb_t - mean_q) ** 2
        sq = jnp.where(mask, sq, 0.0)
        n_selected = jnp.sum(mask.astype(jnp.float32)) * _C
        out_ref[...] = jnp.reshape(jnp.sum(sq) / n_selected, (1, 1))


def kernel(quantized, code, codebook):
    code2d = code.reshape(_N, _K)
    q2d = quantized.reshape(_N, _C)

    out = pl.pallas_call(
        _body,
        grid=(_NBLK,),
        in_specs=[
            pl.BlockSpec((_BLK, _K), lambda i: (i, 0)),
            pl.BlockSpec((_N, _C), lambda i: (0, 0)),
            pl.BlockSpec((_K, _C), lambda i: (0, 0)),
        ],
        out_specs=pl.BlockSpec((1, 1), lambda i: (0, 0)),
        out_shape=jax.ShapeDtypeStruct((1, 1), jnp.float32),
        scratch_shapes=[pltpu.VMEM((_C + 1, _K), jnp.float32)],
        compiler_params=pltpu.CompilerParams(
            dimension_semantics=("arbitrary",),
        ),
    )(code2d, q2d, codebook)
    return out[0, 0]
